# Initial kernel scaffold; baseline (speedup 1.0000x reference)
import jax
import jax.numpy as jnp
from jax.experimental import pallas as pl


def kernel(x, edge_index, edge_type, hsd, W1, b1, g1, be1, W2, b2, g2, be2, Wroot1, Wr1, bg1, gg1, beg1, Wroot2, Wr2, bg2, gg2, beg2, Wout, bout, Wp1, bp1, Wp2, bp2):
    raise NotImplementedError("write your pallas kernel here")


# speedup vs baseline: 1.4041x; 1.4041x over previous
import functools

import jax
import jax.numpy as jnp
from jax import lax
from jax.experimental import pallas as pl
from jax.experimental.pallas import tpu as pltpu

N = 10000
E = 320000
D = 128
H = 256
OUT = 256
R = 3
TAU = 0.1
EPS = 1e-5

_BN = float(1.0 / (1.0 + EPS) ** 0.5)

ROWS = 2000
NBLK = N // ROWS


def _tc1_body(h0, h1, s0, s1, cnt, wcat, bg1, gg1, beg1, wroot2, wr2, bg2,
              z0_o, z1_o, root2_o):
    h = jnp.concatenate([h0[...], h1[...]], axis=1)
    s = jnp.concatenate([s0[...], s1[...]], axis=2)
    means = s / jnp.maximum(cnt[...], 1.0)
    cat = jnp.concatenate([h, means[0], means[1], means[2]], axis=1)
    z = cat @ wcat[...] + bg1[...]
    z1 = jax.nn.relu(z * _BN * gg1[...] + beg1[...])
    root2_o[...] = z1 @ wroot2[...] + bg2[...]
    zr = jnp.einsum('nh,rho->rno', z1, wr2[...],
                    preferred_element_type=jnp.float32)
    z0_o[...] = zr[:, :, :128]
    z1_o[...] = zr[:, :, 128:]


def _tc1(h0, h1, s0, s1, cnt, wcat, bg1, gg1, beg1, wroot2, wr2, bg2):
    row = lambda i: (i, 0)
    row3 = lambda i: (0, i, 0)
    full = lambda i: (0, 0)
    return pl.pallas_call(
        _tc1_body,
        grid=(NBLK,),
        in_specs=[
            pl.BlockSpec((ROWS, 64), row),
            pl.BlockSpec((ROWS, 64), row),
            pl.BlockSpec((R, ROWS, 64), row3),
            pl.BlockSpec((R, ROWS, 64), row3),
            pl.BlockSpec((R, ROWS, 1), lambda i: (0, i, 0)),
            pl.BlockSpec((2 * D + H, H), full),
            pl.BlockSpec((H,), lambda i: (0,)),
            pl.BlockSpec((H,), lambda i: (0,)),
            pl.BlockSpec((H,), lambda i: (0,)),
            pl.BlockSpec((H, OUT), full),
            pl.BlockSpec((R, H, OUT), lambda i: (0, 0, 0)),
            pl.BlockSpec((OUT,), lambda i: (0,)),
        ],
        out_specs=[
            pl.BlockSpec((R, ROWS, 128), row3),
            pl.BlockSpec((R, ROWS, 128), row3),
            pl.BlockSpec((ROWS, OUT), row),
        ],
        out_shape=[
            jax.ShapeDtypeStruct((R, N, 128), jnp.float32),
            jax.ShapeDtypeStruct((R, N, 128), jnp.float32),
            jax.ShapeDtypeStruct((N, OUT), jnp.float32),
        ],
    )(h0, h1, s0, s1, cnt, wcat, bg1, gg1, beg1, wroot2, wr2, bg2)


def _tc2_body(x, root2, c0, c1, gg2, beg2,
              w1, b1, g1, be1, w2, b2, g2, be2, wout, bout, out_o):
    agg2 = jnp.concatenate([c0[...], c1[...]], axis=1)
    z2 = jax.nn.relu((root2[...] + agg2) * _BN * gg2[...] + beg2[...])
    zm = x[...] @ w1[...] + b1[...]
    zm = jax.nn.relu(zm * _BN * g1[...] + be1[...])
    zm = zm @ w2[...] + b2[...]
    zm = jax.nn.relu(zm * _BN * g2[...] + be2[...])
    wa = wout[...][:H]
    wb = wout[...][H:]
    out_o[...] = zm @ wa + z2 @ wb + bout[...]


def _tc2(x, root2, c0, c1, gg2, beg2, w1, b1, g1, be1, w2, b2, g2, be2,
         wout, bout):
    row = lambda i: (i, 0)
    full = lambda i: (0, 0)
    vec = lambda i: (0,)
    out = pl.pallas_call(
        _tc2_body,
        grid=(NBLK,),
        in_specs=[
            pl.BlockSpec((ROWS, D), row),
            pl.BlockSpec((ROWS, OUT), row),
            pl.BlockSpec((ROWS, 128), row),
            pl.BlockSpec((ROWS, 128), row),
            pl.BlockSpec((OUT,), vec),
            pl.BlockSpec((OUT,), vec),
            pl.BlockSpec((D, H), full),
            pl.BlockSpec((H,), vec),
            pl.BlockSpec((H,), vec),
            pl.BlockSpec((H,), vec),
            pl.BlockSpec((H, OUT), full),
            pl.BlockSpec((OUT,), vec),
            pl.BlockSpec((OUT,), vec),
            pl.BlockSpec((OUT,), vec),
            pl.BlockSpec((2 * OUT, 1), full),
            pl.BlockSpec((1,), vec),
        ],
        out_specs=pl.BlockSpec((ROWS, 1), row),
        out_shape=jax.ShapeDtypeStruct((N, 1), jnp.float32),
    )(x, root2, c0, c1, gg2, beg2, w1, b1, g1, be1, w2, b2, g2, be2,
      wout, bout)
    return out[:, 0]


def _edge_phases_jnp(x, src, dst, etype, hsd):
    esc = jnp.exp(-jnp.abs(hsd[src] - hsd[dst]) / TAU)
    denom = jax.ops.segment_sum(esc, dst, num_segments=N)
    acc = jax.ops.segment_sum(esc[:, None] * x[src], dst, num_segments=N)
    h = x + acc / (denom[:, None] + 1e-16)

    sidx = etype * N + dst
    onehot_sum = jax.ops.segment_sum(h[src], sidx, num_segments=R * N)
    cnt = jax.ops.segment_sum(jnp.ones((E,), jnp.float32), sidx,
                              num_segments=R * N)
    return h, onehot_sum.reshape(R, N, D), cnt.reshape(R, N)


def _phase_c_jnp(z0, z1, src, dst, etype, cnt):
    ztab = jnp.concatenate([z0, z1], axis=-1).reshape(R * N, OUT)
    w = 1.0 / jnp.maximum(cnt.reshape(R * N), 1.0)[etype * N + dst]
    agg2 = jax.ops.segment_sum(w[:, None] * ztab[etype * N + src], dst,
                               num_segments=N)
    return agg2


def kernel(x, edge_index, edge_type, hsd, W1, b1, g1, be1, W2, b2, g2, be2,
           Wroot1, Wr1, bg1, gg1, beg1, Wroot2, Wr2, bg2, gg2, beg2,
           Wout, bout, Wp1, bp1, Wp2, bp2):
    src = edge_index[0]
    dst = edge_index[1]

    h, sums, cnt = _edge_phases_jnp(x, src, dst, edge_type, hsd)
    h0, h1 = h[:, :64], h[:, 64:]
    s0, s1 = sums[:, :, :64], sums[:, :, 64:]

    wcat = jnp.concatenate([Wroot1, Wr1.reshape(R * D, H)], axis=0)
    z0, z1t, root2 = _tc1(h0, h1, s0, s1, cnt.reshape(R, N, 1), wcat, bg1, gg1, beg1,
                          Wroot2, Wr2, bg2)

    agg2 = _phase_c_jnp(z0, z1t, src, dst, edge_type, cnt)
    c0, c1 = agg2[:, :128], agg2[:, 128:]

    return _tc2(x, root2, c0, c1, gg2, beg2, W1, b1, g1, be1,
                W2, b2, g2, be2, Wout, bout)


# speedup vs baseline: 6.5303x; 4.6510x over previous
import functools

import jax
import jax.numpy as jnp
from jax import lax
from jax.experimental import pallas as pl
from jax.experimental.pallas import tpu as pltpu
from jax.experimental.pallas import tpu_sc as plsc

N = 10000
E = 320000
D = 128
H = 256
OUT = 256
R = 3
TAU = 0.1

_BN = float(1.0 / (1.0 + 1e-5) ** 0.5)

PADN = 10240
CHUNK = 80
NSTRIPE = PADN // 16
TCROWS = 1280
TCBLK = PADN // TCROWS

_mesh = functools.partial(
    pl.kernel,
    mesh=plsc.VectorSubcoreMesh(core_axis_name="c", subcore_axis_name="s"),
    compiler_params=pltpu.CompilerParams(use_tc_tiling_on_sc=False),
)


def _pa_body(src_h, dst_h, et_h, hsd_h, x2_h, zr_h, zs_h,
             agg_o, den_o, cnt_o,
             srcb, dstb, etb, sv, dv, escb, sidxb, onesb, rows,
             acc, den_sh, cnt_sh, sem):
    c = lax.axis_index("c")
    s = lax.axis_index("s")
    r0 = s * NSTRIPE

    pltpu.sync_copy(zr_h.at[pl.ds(r0, NSTRIPE)], acc.at[pl.ds(r0, NSTRIPE)])
    pltpu.sync_copy(zs_h.at[pl.ds(r0, NSTRIPE)], den_sh.at[pl.ds(r0, NSTRIPE)])
    pltpu.sync_copy(zs_h.at[pl.ds(s * 3 * NSTRIPE, 3 * NSTRIPE)],
                    cnt_sh.at[pl.ds(s * 3 * NSTRIPE, 3 * NSTRIPE)])
    for k in range(CHUNK // 16):
        onesb[pl.ds(k * 16, 16)] = jnp.ones((16,), jnp.float32)
    plsc.subcore_barrier()

    base0 = (c * 16 + s) * (E // 32)

    def body(i, carry):
        base = base0 + i * CHUNK
        pltpu.sync_copy(src_h.at[pl.ds(base, CHUNK)], srcb)
        pltpu.sync_copy(dst_h.at[pl.ds(base, CHUNK)], dstb)
        pltpu.sync_copy(et_h.at[pl.ds(base, CHUNK)], etb)
        pltpu.async_copy(hsd_h.at[srcb], sv, sem).wait()
        pltpu.async_copy(hsd_h.at[dstb], dv, sem).wait()
        for k in range(CHUNK // 16):
            sl = pl.ds(k * 16, 16)
            esc = jnp.exp(jnp.abs(sv[sl] - dv[sl]) * (-1.0 / TAU))
            escb[sl] = esc
            sidxb[sl] = etb[sl] * PADN + dstb[sl]
        pltpu.sync_copy(escb, den_sh.at[dstb], add=True)
        pltpu.sync_copy(onesb, cnt_sh.at[sidxb], add=True)
        pltpu.async_copy(x2_h.at[srcb], rows, sem).wait()
        for kg in range(CHUNK // 16):
            ev = escb[pl.ds(kg * 16, 16)]
            for jl in range(16):
                j = kg * 16 + jl
                e = ev[jl]
                for m in range(D // 16):
                    cs = pl.ds(m * 16, 16)
                    rows[j, cs] = rows[j, cs] * e
        pltpu.sync_copy(rows, acc.at[dstb], add=True)
        return carry

    lax.fori_loop(0, (E // 32) // CHUNK, body, 0)
    plsc.subcore_barrier()

    pltpu.sync_copy(acc.at[pl.ds(r0, NSTRIPE)],
                    agg_o.at[pl.ds(c * PADN + r0, NSTRIPE)])
    pltpu.sync_copy(den_sh.at[pl.ds(r0, NSTRIPE)],
                    den_o.at[pl.ds(c * PADN + r0, NSTRIPE)])
    pltpu.sync_copy(cnt_sh.at[pl.ds(s * 3 * NSTRIPE, 3 * NSTRIPE)],
                    cnt_o.at[pl.ds(c * 3 * PADN + s * 3 * NSTRIPE,
                                   3 * NSTRIPE)])


def _phase_a(src, dst, et, hsd, x2, zr, zs):
    f = _mesh(
        _pa_body,
        out_type=[
            jax.ShapeDtypeStruct((2 * PADN, D), jnp.float32),
            jax.ShapeDtypeStruct((2 * PADN,), jnp.float32),
            jax.ShapeDtypeStruct((2 * 3 * PADN,), jnp.float32),
        ],
        scratch_types=[
            pltpu.VMEM((CHUNK,), jnp.int32),
            pltpu.VMEM((CHUNK,), jnp.int32),
            pltpu.VMEM((CHUNK,), jnp.int32),
            pltpu.VMEM((CHUNK,), jnp.float32),
            pltpu.VMEM((CHUNK,), jnp.float32),
            pltpu.VMEM((CHUNK,), jnp.float32),
            pltpu.VMEM((CHUNK,), jnp.int32),
            pltpu.VMEM((CHUNK,), jnp.float32),
            pltpu.VMEM((CHUNK, D), jnp.float32),
            pltpu.VMEM_SHARED((PADN, D), jnp.float32),
            pltpu.VMEM_SHARED((PADN,), jnp.float32),
            pltpu.VMEM_SHARED((3 * PADN,), jnp.float32),
            pltpu.SemaphoreType.DMA,
        ],
    )
    return f(src, dst, et, hsd, x2, zr, zs)


def _wg_body(save_w, src_h, dst_h, et_h, tab_h, wsrc_h, zr_h, agg_o, w_o,
             srcb, dstb, etb, gidxb, widxb, wb, rows, acc, sem):
    c = lax.axis_index("c")
    s = lax.axis_index("s")
    r0 = s * NSTRIPE

    pltpu.sync_copy(zr_h.at[pl.ds(r0, NSTRIPE)], acc.at[pl.ds(r0, NSTRIPE)])
    plsc.subcore_barrier()

    base0 = s * (E // 16)
    coff = c * 3 * PADN

    def body(i, carry):
        base = base0 + i * CHUNK
        pltpu.sync_copy(src_h.at[pl.ds(base, CHUNK)], srcb)
        pltpu.sync_copy(dst_h.at[pl.ds(base, CHUNK)], dstb)
        pltpu.sync_copy(et_h.at[pl.ds(base, CHUNK)], etb)
        if save_w:
            for k in range(CHUNK // 16):
                sl = pl.ds(k * 16, 16)
                gidxb[sl] = coff + etb[sl] * PADN + srcb[sl]
                widxb[sl] = etb[sl] * PADN + dstb[sl]
            pltpu.async_copy(wsrc_h.at[widxb], wb, sem).wait()

            @pl.when(c == 0)
            def _():
                pltpu.sync_copy(wb, w_o.at[pl.ds(base, CHUNK)])
        else:
            for k in range(CHUNK // 16):
                sl = pl.ds(k * 16, 16)
                gidxb[sl] = coff + etb[sl] * PADN + srcb[sl]
            pltpu.sync_copy(wsrc_h.at[pl.ds(base, CHUNK)], wb)
        pltpu.async_copy(tab_h.at[gidxb], rows, sem).wait()
        for kg in range(CHUNK // 16):
            ev = wb[pl.ds(kg * 16, 16)]
            for jl in range(16):
                j = kg * 16 + jl
                e = ev[jl]
                for m in range(128 // 16):
                    cs = pl.ds(m * 16, 16)
                    rows[j, cs] = rows[j, cs] * e
        pltpu.sync_copy(rows, acc.at[dstb], add=True)
        return carry

    lax.fori_loop(0, (E // 16) // CHUNK, body, 0)
    plsc.subcore_barrier()

    pltpu.sync_copy(acc.at[pl.ds(r0, NSTRIPE)],
                    agg_o.at[pl.ds(c * PADN + r0, NSTRIPE)])


def _phase_wg(save_w, src, dst, et, tab, wsrc, zr):
    f = _mesh(
        functools.partial(_wg_body, save_w),
        out_type=[
            jax.ShapeDtypeStruct((2 * PADN, 128), jnp.float32),
            jax.ShapeDtypeStruct((E,), jnp.float32),
        ],
        scratch_types=[
            pltpu.VMEM((CHUNK,), jnp.int32),
            pltpu.VMEM((CHUNK,), jnp.int32),
            pltpu.VMEM((CHUNK,), jnp.int32),
            pltpu.VMEM((CHUNK,), jnp.int32),
            pltpu.VMEM((CHUNK,), jnp.int32),
            pltpu.VMEM((CHUNK,), jnp.float32),
            pltpu.VMEM((CHUNK, 128), jnp.float32),
            pltpu.VMEM_SHARED((PADN, 128), jnp.float32),
            pltpu.SemaphoreType.DMA,
        ],
    )
    return f(src, dst, et, tab, wsrc, zr)


def _tcm_body(x2, a0, a1, d_r, c_r, wr1, htab_o, inv_o, h1_o):
    pid = pl.program_id(0)
    agg = a0[...] + a1[...]
    d0 = d_r[pl.ds(pid * TCROWS, TCROWS)]
    d1 = d_r[pl.ds(PADN + pid * TCROWS, TCROWS)]
    den = (d0 + d1).reshape(TCROWS, 1)
    htab = x2[...] + agg / (den + 1e-16)
    htab_o[...] = htab
    cnt = (c_r[0:3, pl.ds(pid * TCROWS, TCROWS)]
           + c_r[3:6, pl.ds(pid * TCROWS, TCROWS)])
    inv_o[...] = 1.0 / jnp.maximum(cnt, 1.0)
    h1 = jnp.einsum('nd,rdh->rnh', htab, wr1[...],
                    preferred_element_type=jnp.float32)
    h1_o[0] = h1[:, :, :128]
    h1_o[1] = h1[:, :, 128:]


def _tc_mid(x2, aggp, denp, cntp6, wr1):
    row = lambda i: (i, 0)
    return pl.pallas_call(
        _tcm_body,
        grid=(TCBLK,),
        in_specs=[
            pl.BlockSpec((TCROWS, D), row),
            pl.BlockSpec((TCROWS, D), row),
            pl.BlockSpec((TCROWS, D), lambda i: (TCBLK + i, 0)),
            pl.BlockSpec((2 * PADN,), lambda i: (0,)),
            pl.BlockSpec((2 * R, PADN), lambda i: (0, 0)),
            pl.BlockSpec((R, D, H), lambda i: (0, 0, 0)),
        ],
        out_specs=[
            pl.BlockSpec((TCROWS, D), row),
            pl.BlockSpec((R, TCROWS), lambda i: (0, i)),
            pl.BlockSpec((2, R, TCROWS, 128), lambda i: (0, 0, i, 0)),
        ],
        out_shape=[
            jax.ShapeDtypeStruct((PADN, D), jnp.float32),
            jax.ShapeDtypeStruct((R, PADN), jnp.float32),
            jax.ShapeDtypeStruct((2, R, PADN, 128), jnp.float32),
        ],
    )(x2, aggp, aggp, denp, cntp6, wr1)


def _tc1_body(htab, a0, a1, wroot1, bg1, gg1, beg1, wroot2, wr2, bg2,
              z_o, root2_o):
    agg1 = jnp.concatenate([a0[...], a1[...]], axis=1)
    z = htab[...] @ wroot1[...] + bg1[...] + agg1
    z1 = jax.nn.relu(z * _BN * gg1[...] + beg1[...])
    root2_o[...] = z1 @ wroot2[...] + bg2[...]
    zr = jnp.einsum('nh,rho->rno', z1, wr2[...],
                    preferred_element_type=jnp.float32)
    z_o[0] = zr[:, :, :128]
    z_o[1] = zr[:, :, 128:]


def _tc1(htab, agg1p, wroot1, bg1, gg1, beg1, wroot2, wr2, bg2):
    row = lambda i: (i, 0)
    vec = lambda i: (0,)
    return pl.pallas_call(
        _tc1_body,
        grid=(TCBLK,),
        in_specs=[
            pl.BlockSpec((TCROWS, D), row),
            pl.BlockSpec((TCROWS, 128), row),
            pl.BlockSpec((TCROWS, 128), lambda i: (TCBLK + i, 0)),
            pl.BlockSpec((D, H), lambda i: (0, 0)),
            pl.BlockSpec((H,), vec),
            pl.BlockSpec((H,), vec),
            pl.BlockSpec((H,), vec),
            pl.BlockSpec((H, OUT), lambda i: (0, 0)),
            pl.BlockSpec((R, H, OUT), lambda i: (0, 0, 0)),
            pl.BlockSpec((OUT,), vec),
        ],
        out_specs=[
            pl.BlockSpec((2, R, TCROWS, 128), lambda i: (0, 0, i, 0)),
            pl.BlockSpec((TCROWS, OUT), row),
        ],
        out_shape=[
            jax.ShapeDtypeStruct((2, R, PADN, 128), jnp.float32),
            jax.ShapeDtypeStruct((PADN, OUT), jnp.float32),
        ],
    )(htab, agg1p, agg1p, wroot1, bg1, gg1, beg1, wroot2, wr2, bg2)


def _tc2_body(x2, root2, c0, c1, gg2, beg2,
              w1, b1, g1, be1, w2, b2, g2, be2, wout, bout, out_o):
    agg2 = jnp.concatenate([c0[...], c1[...]], axis=1)
    z2 = jax.nn.relu((root2[...] + agg2) * _BN * gg2[...] + beg2[...])
    zm = x2[...] @ w1[...] + b1[...]
    zm = jax.nn.relu(zm * _BN * g1[...] + be1[...])
    zm = zm @ w2[...] + b2[...]
    zm = jax.nn.relu(zm * _BN * g2[...] + be2[...])
    wa = wout[...][:H]
    wb = wout[...][H:]
    out_o[...] = zm @ wa + z2 @ wb + bout[...]


def _tc2(x2, root2, agg2p, gg2, beg2, w1, b1, g1, be1, w2, b2, g2, be2,
         wout, bout):
    row = lambda i: (i, 0)
    vec = lambda i: (0,)
    out = pl.pallas_call(
        _tc2_body,
        grid=(TCBLK,),
        in_specs=[
            pl.BlockSpec((TCROWS, D), row),
            pl.BlockSpec((TCROWS, OUT), row),
            pl.BlockSpec((TCROWS, 128), row),
            pl.BlockSpec((TCROWS, 128), lambda i: (TCBLK + i, 0)),
            pl.BlockSpec((OUT,), vec),
            pl.BlockSpec((OUT,), vec),
            pl.BlockSpec((D, H), lambda i: (0, 0)),
            pl.BlockSpec((H,), vec),
            pl.BlockSpec((H,), vec),
            pl.BlockSpec((H,), vec),
            pl.BlockSpec((H, OUT), lambda i: (0, 0)),
            pl.BlockSpec((OUT,), vec),
            pl.BlockSpec((OUT,), vec),
            pl.BlockSpec((OUT,), vec),
            pl.BlockSpec((2 * OUT, 1), lambda i: (0, 0)),
            pl.BlockSpec((1,), vec),
        ],
        out_specs=pl.BlockSpec((TCROWS, 1), row),
        out_shape=jax.ShapeDtypeStruct((PADN, 1), jnp.float32),
    )(x2, root2, agg2p, agg2p, gg2, beg2, w1, b1, g1, be1, w2, b2, g2, be2,
      wout, bout)
    return out[:N, 0]


def kernel(x, edge_index, edge_type, hsd, W1, b1, g1, be1, W2, b2, g2, be2,
           Wroot1, Wr1, bg1, gg1, beg1, Wroot2, Wr2, bg2, gg2, beg2,
           Wout, bout, Wp1, bp1, Wp2, bp2):
    src = edge_index[0]
    dst = edge_index[1]

    x2 = jnp.zeros((PADN, D), jnp.float32).at[:N].set(x)
    zr = jnp.zeros((PADN, 128), jnp.float32)
    zs = jnp.zeros((3 * PADN,), jnp.float32)

    aggp, denp, cntp = _phase_a(src, dst, edge_type, hsd, x2, zr, zs)

    htab, invcnt, h1tab = _tc_mid(x2, aggp, denp, cntp.reshape(2 * R, PADN),
                                  Wr1)

    agg1p, wsave = _phase_wg(True, src, dst, edge_type,
                             h1tab.reshape(2 * R * PADN, 128),
                             invcnt.reshape(R * PADN), zr)

    ztab, root2 = _tc1(htab, agg1p, Wroot1, bg1, gg1, beg1, Wroot2, Wr2, bg2)

    agg2p, _ = _phase_wg(False, src, dst, edge_type,
                         ztab.reshape(2 * R * PADN, 128), wsave, zr)

    return _tc2(x2, root2, agg2p, gg2, beg2, W1, b1, g1, be1,
                W2, b2, g2, be2, Wout, bout)


# speedup vs baseline: 9.5141x; 1.4569x over previous
import functools

import jax
import jax.numpy as jnp
from jax import lax
from jax.experimental import pallas as pl
from jax.experimental.pallas import tpu as pltpu
from jax.experimental.pallas import tpu_sc as plsc

N = 10000
E = 320000
D = 128
H = 256
OUT = 256
R = 3
TAU = 0.1

_BN = float(1.0 / (1.0 + 1e-5) ** 0.5)

PADN = 10240
CHUNK = 80
NSTRIPE = PADN // 16
TCROWS = 1280
TCBLK = PADN // TCROWS

_mesh = functools.partial(
    pl.kernel,
    mesh=plsc.VectorSubcoreMesh(core_axis_name="c", subcore_axis_name="s"),
    compiler_params=pltpu.CompilerParams(use_tc_tiling_on_sc=False),
)


def _pa_body(src_h, dst_h, et_h, hsd_h, x2_h, zr_h, zs_h,
             agg_o, den_o, cnt_o,
             srcb, dstb, etb, sv, dv, escb, sidxb, onesb, rows,
             acc, den_sh, cnt_sh, sem, sem2):
    c = lax.axis_index("c")
    s = lax.axis_index("s")
    r0 = s * NSTRIPE

    pltpu.sync_copy(zr_h.at[pl.ds(r0, NSTRIPE)], acc.at[pl.ds(r0, NSTRIPE)])
    pltpu.sync_copy(zs_h.at[pl.ds(r0, NSTRIPE)], den_sh.at[pl.ds(r0, NSTRIPE)])
    pltpu.sync_copy(zs_h.at[pl.ds(s * 3 * NSTRIPE, 3 * NSTRIPE)],
                    cnt_sh.at[pl.ds(s * 3 * NSTRIPE, 3 * NSTRIPE)])
    for k in range(CHUNK // 16):
        onesb[pl.ds(k * 16, 16)] = jnp.ones((16,), jnp.float32)
    plsc.subcore_barrier()

    base0 = (c * 16 + s) * (E // 32)
    nch = (E // 32) // CHUNK

    def load_edges(j, b):
        base = base0 + j * CHUNK
        pltpu.sync_copy(src_h.at[pl.ds(base, CHUNK)], srcb[b])
        pltpu.sync_copy(dst_h.at[pl.ds(base, CHUNK)], dstb[b])
        pltpu.sync_copy(et_h.at[pl.ds(base, CHUNK)], etb[b])

    def issue(b):
        pltpu.async_copy(hsd_h.at[srcb[b]], sv[b], sem[b])
        pltpu.async_copy(hsd_h.at[dstb[b]], dv[b], sem[b])
        pltpu.async_copy(x2_h.at[srcb[b]], rows[b], sem2[b])

    def wait(b):
        pltpu.make_async_copy(hsd_h.at[srcb[b]], sv[b], sem[b]).wait()
        pltpu.make_async_copy(hsd_h.at[dstb[b]], dv[b], sem[b]).wait()
        pltpu.make_async_copy(x2_h.at[srcb[b]], rows[b], sem2[b]).wait()

    def process(b):
        for k in range(CHUNK // 16):
            sl = pl.ds(k * 16, 16)
            esc = jnp.exp(jnp.abs(sv[b][sl] - dv[b][sl]) * (-1.0 / TAU))
            escb[b][sl] = esc
            sidxb[b][sl] = etb[b][sl] * PADN + dstb[b][sl]
        pltpu.sync_copy(escb[b], den_sh.at[dstb[b]], add=True)
        pltpu.sync_copy(onesb, cnt_sh.at[sidxb[b]], add=True)
        for kg in range(CHUNK // 16):
            ev = escb[b][pl.ds(kg * 16, 16)]
            for jl in range(16):
                j = kg * 16 + jl
                e = ev[jl]
                for m in range(D // 16):
                    cs = pl.ds(m * 16, 16)
                    rows[b][j, cs] = rows[b][j, cs] * e
        pltpu.sync_copy(rows[b], acc.at[dstb[b]], add=True)

    load_edges(0, 0)
    issue(0)

    def body(i, carry):
        j0 = i * 2

        @pl.when(j0 + 1 < nch)
        def _():
            load_edges(j0 + 1, 1)
            issue(1)

        wait(0)
        process(0)

        @pl.when(j0 + 2 < nch)
        def _():
            load_edges(j0 + 2, 0)
            issue(0)

        @pl.when(j0 + 1 < nch)
        def _():
            wait(1)
            process(1)

        return carry

    lax.fori_loop(0, (nch + 1) // 2, body, 0)
    plsc.subcore_barrier()

    pltpu.sync_copy(acc.at[pl.ds(r0, NSTRIPE)],
                    agg_o.at[pl.ds(c * PADN + r0, NSTRIPE)])
    pltpu.sync_copy(den_sh.at[pl.ds(r0, NSTRIPE)],
                    den_o.at[pl.ds(c * PADN + r0, NSTRIPE)])
    pltpu.sync_copy(cnt_sh.at[pl.ds(s * 3 * NSTRIPE, 3 * NSTRIPE)],
                    cnt_o.at[pl.ds(c * 3 * PADN + s * 3 * NSTRIPE,
                                   3 * NSTRIPE)])


def _phase_a(src, dst, et, hsd, x2, zr, zs):
    ib = lambda: pltpu.VMEM((CHUNK,), jnp.int32)
    fb = lambda: pltpu.VMEM((CHUNK,), jnp.float32)
    f = _mesh(
        _pa_body,
        out_type=[
            jax.ShapeDtypeStruct((2 * PADN, D), jnp.float32),
            jax.ShapeDtypeStruct((2 * PADN,), jnp.float32),
            jax.ShapeDtypeStruct((2 * 3 * PADN,), jnp.float32),
        ],
        scratch_types=[
            [ib(), ib()], [ib(), ib()], [ib(), ib()],
            [fb(), fb()], [fb(), fb()], [fb(), fb()],
            [ib(), ib()],
            fb(),
            [pltpu.VMEM((CHUNK, D), jnp.float32),
             pltpu.VMEM((CHUNK, D), jnp.float32)],
            pltpu.VMEM_SHARED((PADN, D), jnp.float32),
            pltpu.VMEM_SHARED((PADN,), jnp.float32),
            pltpu.VMEM_SHARED((3 * PADN,), jnp.float32),
            [pltpu.SemaphoreType.DMA, pltpu.SemaphoreType.DMA],
            [pltpu.SemaphoreType.DMA, pltpu.SemaphoreType.DMA],
        ],
    )
    return f(src, dst, et, hsd, x2, zr, zs)


def _seg_body(npass, src_h, dst_h, et_h, tab_h, z64_h, sums_o,
              srcb, dstb, etb, gidxb, sidxb, rows, acc, sem):
    c = lax.axis_index("c")
    s = lax.axis_index("s")
    z0 = s * (3 * N // 16)

    base0 = s * (E // 16)
    nch = (E // 16) // CHUNK

    def load_edges(j, b):
        base = base0 + j * CHUNK
        pltpu.sync_copy(src_h.at[pl.ds(base, CHUNK)], srcb[b])
        pltpu.sync_copy(dst_h.at[pl.ds(base, CHUNK)], dstb[b])
        pltpu.sync_copy(et_h.at[pl.ds(base, CHUNK)], etb[b])

    for p in range(npass):
        qoff = (c * npass + p) * PADN

        pltpu.sync_copy(z64_h.at[pl.ds(z0, 3 * N // 16)],
                        acc.at[pl.ds(z0, 3 * N // 16)])
        plsc.subcore_barrier()

        def prep(b, qo):
            for k in range(CHUNK // 16):
                sl = pl.ds(k * 16, 16)
                gidxb[b][sl] = qo + srcb[b][sl]
                sidxb[b][sl] = etb[b][sl] * N + dstb[b][sl]

        def issue(b):
            pltpu.async_copy(tab_h.at[gidxb[b]], rows[b], sem[b])

        def wait_scatter(b):
            pltpu.make_async_copy(tab_h.at[gidxb[b]], rows[b], sem[b]).wait()
            pltpu.sync_copy(rows[b], acc.at[sidxb[b]], add=True)

        load_edges(0, 0)
        prep(0, qoff)
        issue(0)

        def body(i, carry):
            j0 = i * 2

            @pl.when(j0 + 1 < nch)
            def _():
                load_edges(j0 + 1, 1)
                prep(1, qoff)
                issue(1)

            wait_scatter(0)

            @pl.when(j0 + 2 < nch)
            def _():
                load_edges(j0 + 2, 0)
                prep(0, qoff)
                issue(0)

            @pl.when(j0 + 1 < nch)
            def _():
                wait_scatter(1)

            return carry

        lax.fori_loop(0, (nch + 1) // 2, body, 0)
        plsc.subcore_barrier()

        for r in range(R):
            pltpu.sync_copy(
                acc.at[pl.ds(r * N + s * (N // 16), N // 16)],
                sums_o.at[pl.ds((c * npass + p) * 3 * PADN + r * PADN
                                + s * (N // 16), N // 16)])
        if p + 1 < npass:
            plsc.subcore_barrier()


def _phase_seg(npass, src, dst, et, tab, z64):
    ib = lambda: pltpu.VMEM((CHUNK,), jnp.int32)
    f = _mesh(
        functools.partial(_seg_body, npass),
        out_type=jax.ShapeDtypeStruct((2 * npass * 3 * PADN, 64),
                                      jnp.float32),
        scratch_types=[
            [ib(), ib()], [ib(), ib()], [ib(), ib()],
            [ib(), ib()], [ib(), ib()],
            [pltpu.VMEM((CHUNK, 64), jnp.float32),
             pltpu.VMEM((CHUNK, 64), jnp.float32)],
            pltpu.VMEM_SHARED((3 * N, 64), jnp.float32),
            [pltpu.SemaphoreType.DMA, pltpu.SemaphoreType.DMA],
        ],
    )
    return f(src, dst, et, tab, z64)


def _tcm_body(x2, a0, a1, d_r, htab_o, h64_o):
    pid = pl.program_id(0)
    agg = a0[...] + a1[...]
    d0 = d_r[pl.ds(pid * TCROWS, TCROWS)]
    d1 = d_r[pl.ds(PADN + pid * TCROWS, TCROWS)]
    den = (d0 + d1).reshape(TCROWS, 1)
    htab = x2[...] + agg / (den + 1e-16)
    htab_o[...] = htab
    h64_o[0] = htab[:, :64]
    h64_o[1] = htab[:, 64:]


def _tc_mid(x2, aggp, denp):
    row = lambda i: (i, 0)
    return pl.pallas_call(
        _tcm_body,
        grid=(TCBLK,),
        in_specs=[
            pl.BlockSpec((TCROWS, D), row),
            pl.BlockSpec((TCROWS, D), row),
            pl.BlockSpec((TCROWS, D), lambda i: (TCBLK + i, 0)),
            pl.BlockSpec((2 * PADN,), lambda i: (0,)),
        ],
        out_specs=[
            pl.BlockSpec((TCROWS, D), row),
            pl.BlockSpec((2, TCROWS, 64), lambda i: (0, i, 0)),
        ],
        out_shape=[
            jax.ShapeDtypeStruct((PADN, D), jnp.float32),
            jax.ShapeDtypeStruct((2, PADN, 64), jnp.float32),
        ],
    )(x2, aggp, aggp, denp)


def _tc1_body(htab, s1, c_r, wroot1, wr1, bg1, gg1, beg1, wroot2, bg2,
              z64_o, root2_o):
    pid = pl.program_id(0)
    cnt = (c_r[0:3, pl.ds(pid * TCROWS, TCROWS)]
           + c_r[3:6, pl.ds(pid * TCROWS, TCROWS)])
    inv = (1.0 / jnp.maximum(cnt, 1.0))[:, :, None]
    s1v = s1[...]
    means = jnp.concatenate([s1v[0:3], s1v[3:6]], axis=2) * inv
    w1v = wr1[...]
    agg1 = (means[0] @ w1v[0] + means[1] @ w1v[1] + means[2] @ w1v[2])
    z = htab[...] @ wroot1[...] + bg1[...] + agg1
    z1 = jax.nn.relu(z * _BN * gg1[...] + beg1[...])
    root2_o[...] = z1 @ wroot2[...] + bg2[...]
    for q in range(4):
        z64_o[q] = z1[:, q * 64:(q + 1) * 64]


def _tc1(htab, sums1, cntp6, wroot1, wr1, bg1, gg1, beg1, wroot2, bg2):
    row = lambda i: (i, 0)
    vec = lambda i: (0,)
    return pl.pallas_call(
        _tc1_body,
        grid=(TCBLK,),
        in_specs=[
            pl.BlockSpec((TCROWS, D), row),
            pl.BlockSpec((6, TCROWS, 64), lambda i: (0, i, 0)),
            pl.BlockSpec((2 * R, PADN), lambda i: (0, 0)),
            pl.BlockSpec((D, H), lambda i: (0, 0)),
            pl.BlockSpec((R, D, H), lambda i: (0, 0, 0)),
            pl.BlockSpec((H,), vec),
            pl.BlockSpec((H,), vec),
            pl.BlockSpec((H,), vec),
            pl.BlockSpec((H, OUT), lambda i: (0, 0)),
            pl.BlockSpec((OUT,), vec),
        ],
        out_specs=[
            pl.BlockSpec((4, TCROWS, 64), lambda i: (0, i, 0)),
            pl.BlockSpec((TCROWS, OUT), row),
        ],
        out_shape=[
            jax.ShapeDtypeStruct((4, PADN, 64), jnp.float32),
            jax.ShapeDtypeStruct((PADN, OUT), jnp.float32),
        ],
    )(htab, sums1, cntp6, wroot1, wr1, bg1, gg1, beg1, wroot2, bg2)


def _tc2_body(x2, root2, s2, c_r, wr2, gg2, beg2,
              w1, b1, g1, be1, w2, b2, g2, be2, wout, bout, out_o):
    pid = pl.program_id(0)
    cnt = (c_r[0:3, pl.ds(pid * TCROWS, TCROWS)]
           + c_r[3:6, pl.ds(pid * TCROWS, TCROWS)])
    inv = (1.0 / jnp.maximum(cnt, 1.0))[:, :, None]
    s2v = s2[...]
    means = jnp.concatenate([s2v[0:3], s2v[3:6], s2v[6:9], s2v[9:12]],
                            axis=2) * inv
    w2v = wr2[...]
    agg2 = (means[0] @ w2v[0] + means[1] @ w2v[1] + means[2] @ w2v[2])
    z2 = jax.nn.relu((root2[...] + agg2) * _BN * gg2[...] + beg2[...])
    zm = x2[...] @ w1[...] + b1[...]
    zm = jax.nn.relu(zm * _BN * g1[...] + be1[...])
    zm = zm @ w2[...] + b2[...]
    zm = jax.nn.relu(zm * _BN * g2[...] + be2[...])
    wa = wout[...][:H]
    wb = wout[...][H:]
    out_o[...] = zm @ wa + z2 @ wb + bout[...]


def _tc2(x2, root2, sums2, cntp6, wr2, gg2, beg2,
         w1, b1, g1, be1, w2, b2, g2, be2, wout, bout):
    row = lambda i: (i, 0)
    vec = lambda i: (0,)
    out = pl.pallas_call(
        _tc2_body,
        grid=(TCBLK,),
        in_specs=[
            pl.BlockSpec((TCROWS, D), row),
            pl.BlockSpec((TCROWS, OUT), row),
            pl.BlockSpec((12, TCROWS, 64), lambda i: (0, i, 0)),
            pl.BlockSpec((2 * R, PADN), lambda i: (0, 0)),
            pl.BlockSpec((R, H, OUT), lambda i: (0, 0, 0)),
            pl.BlockSpec((OUT,), vec),
            pl.BlockSpec((OUT,), vec),
            pl.BlockSpec((D, H), lambda i: (0, 0)),
            pl.BlockSpec((H,), vec),
            pl.BlockSpec((H,), vec),
            pl.BlockSpec((H,), vec),
            pl.BlockSpec((H, OUT), lambda i: (0, 0)),
            pl.BlockSpec((OUT,), vec),
            pl.BlockSpec((OUT,), vec),
            pl.BlockSpec((OUT,), vec),
            pl.BlockSpec((2 * OUT, 1), lambda i: (0, 0)),
            pl.BlockSpec((1,), vec),
        ],
        out_specs=pl.BlockSpec((TCROWS, 1), row),
        out_shape=jax.ShapeDtypeStruct((PADN, 1), jnp.float32),
    )(x2, root2, sums2, cntp6, wr2, gg2, beg2, w1, b1, g1, be1,
      w2, b2, g2, be2, wout, bout)
    return out[:N, 0]


def kernel(x, edge_index, edge_type, hsd, W1, b1, g1, be1, W2, b2, g2, be2,
           Wroot1, Wr1, bg1, gg1, beg1, Wroot2, Wr2, bg2, gg2, beg2,
           Wout, bout, Wp1, bp1, Wp2, bp2):
    src = edge_index[0]
    dst = edge_index[1]

    x2 = jnp.zeros((PADN, D), jnp.float32).at[:N].set(x)
    zr = jnp.zeros((PADN, 128), jnp.float32)
    zs = jnp.zeros((3 * PADN,), jnp.float32)
    zq = jnp.zeros((3 * PADN, 64), jnp.float32)

    aggp, denp, cntp = _phase_a(src, dst, edge_type, hsd, x2, zr, zs)
    cntp6 = cntp.reshape(2 * R, PADN)

    htab, h64 = _tc_mid(x2, aggp, denp)

    sums1 = _phase_seg(1, src, dst, edge_type,
                       h64.reshape(2 * PADN, 64), zq)

    ztab, root2 = _tc1(htab, sums1.reshape(6, PADN, 64), cntp6,
                       Wroot1, Wr1, bg1, gg1, beg1, Wroot2, bg2)

    sums2 = _phase_seg(2, src, dst, edge_type,
                       ztab.reshape(4 * PADN, 64), zq)

    return _tc2(x2, root2, sums2.reshape(12, PADN, 64), cntp6, Wr2,
                gg2, beg2, W1, b1, g1, be1, W2, b2, g2, be2, Wout, bout)


# speedup vs baseline: 17.6725x; 1.8575x over previous
import functools

import jax
import jax.numpy as jnp
from jax import lax
from jax.experimental import pallas as pl
from jax.experimental.pallas import tpu as pltpu
from jax.experimental.pallas import tpu_sc as plsc

N = 10000
E = 320000
D = 128
H = 256
OUT = 256
R = 3
TAU = 0.1

_BN = float(1.0 / (1.0 + 1e-5) ** 0.5)

PADN = 10240
CHUNK = 80
NSTRIPE = PADN // 16
TCROWS = 1280
TCBLK = PADN // TCROWS

_mesh = functools.partial(
    pl.kernel,
    mesh=plsc.VectorSubcoreMesh(core_axis_name="c", subcore_axis_name="s"),
    compiler_params=pltpu.CompilerParams(use_tc_tiling_on_sc=False),
)


def _pa_body(ei_h, hsd_h, x2_h, zr_h, zs_h,
             agg_o, den_o, cnt_o,
             eb, srcb, dstb, sv, dv, escb, sidxb, onesb, rows,
             acc, den_sh, cnt_sh, esem, sem, sem2, ssem):
    c = lax.axis_index("c")
    s = lax.axis_index("s")
    r0 = s * NSTRIPE
    nch = (E // 32) // CHUNK
    gc0 = (c * 16 + s) * nch

    pltpu.sync_copy(zr_h.at[pl.ds(r0, NSTRIPE)], acc.at[pl.ds(r0, NSTRIPE)])
    pltpu.sync_copy(zs_h.at[pl.ds(r0, NSTRIPE)], den_sh.at[pl.ds(r0, NSTRIPE)])
    pltpu.sync_copy(zs_h.at[pl.ds(s * 3 * NSTRIPE, 3 * NSTRIPE)],
                    cnt_sh.at[pl.ds(s * 3 * NSTRIPE, 3 * NSTRIPE)])
    for k in range(CHUNK // 16):
        onesb[pl.ds(k * 16, 16)] = jnp.ones((16,), jnp.float32)
    plsc.subcore_barrier()

    def load_edges(j, b):
        pltpu.async_copy(ei_h.at[pl.ds((gc0 + j) * 3 * CHUNK, 3 * CHUNK)],
                         eb[b], esem[b])

    def prep_issue(b):
        pltpu.make_async_copy(
            ei_h.at[pl.ds(0, 3 * CHUNK)], eb[b], esem[b]).wait()
        for k in range(CHUNK // 16):
            sl = pl.ds(k * 16, 16)
            srcv = eb[b][sl]
            dstv = eb[b][pl.ds(CHUNK + k * 16, 16)]
            srcb[b][sl] = srcv
            dstb[b][sl] = dstv
            sidxb[b][sl] = eb[b][pl.ds(2 * CHUNK + k * 16, 16)] * PADN + dstv
        pltpu.async_copy(hsd_h.at[srcb[b]], sv[b], sem[b])
        pltpu.async_copy(hsd_h.at[dstb[b]], dv[b], sem[b])
        pltpu.async_copy(x2_h.at[srcb[b]], rows[b], sem2[b])

    def wait_in(b):
        pltpu.make_async_copy(hsd_h.at[srcb[b]], sv[b], sem[b]).wait()
        pltpu.make_async_copy(hsd_h.at[dstb[b]], dv[b], sem[b]).wait()
        pltpu.make_async_copy(x2_h.at[srcb[b]], rows[b], sem2[b]).wait()

    def process(b):
        for k in range(CHUNK // 16):
            sl = pl.ds(k * 16, 16)
            esc = jnp.exp(jnp.abs(sv[b][sl] - dv[b][sl]) * (-1.0 / TAU))
            escb[b][sl] = esc
        pltpu.sync_copy(escb[b], den_sh.at[dstb[b]], add=True)
        pltpu.sync_copy(onesb, cnt_sh.at[sidxb[b]], add=True)
        for kg in range(CHUNK // 16):
            ev = escb[b][pl.ds(kg * 16, 16)]
            for jl in range(16):
                j = kg * 16 + jl
                e = ev[jl]
                for m in range(D // 16):
                    cs = pl.ds(m * 16, 16)
                    rows[b][j, cs] = rows[b][j, cs] * e
        pltpu.async_copy(rows[b], acc.at[dstb[b]], ssem[b], add=True)

    def wait_sc(b):
        pltpu.make_async_copy(rows[b], acc.at[dstb[b]], ssem[b]).wait()

    load_edges(0, 0)
    load_edges(1, 1)
    prep_issue(0)

    def body(i, carry):
        j0 = i * 2

        @pl.when(i > 0)
        def _():
            wait_sc(1)

        @pl.when(j0 + 1 < nch)
        def _():
            prep_issue(1)

        @pl.when(j0 + 2 < nch)
        def _():
            load_edges(j0 + 2, 0)

        wait_in(0)
        process(0)

        @pl.when(j0 + 2 < nch)
        def _():
            wait_sc(0)
            prep_issue(0)

        @pl.when(j0 + 3 < nch)
        def _():
            load_edges(j0 + 3, 1)

        @pl.when(j0 + 1 < nch)
        def _():
            wait_in(1)
            process(1)

        return carry

    lax.fori_loop(0, (nch + 1) // 2, body, 0)
    wait_sc(0)
    if nch % 2 == 0:
        wait_sc(1)
    plsc.subcore_barrier()

    pltpu.sync_copy(acc.at[pl.ds(r0, NSTRIPE)],
                    agg_o.at[pl.ds(c * PADN + r0, NSTRIPE)])
    pltpu.sync_copy(den_sh.at[pl.ds(r0, NSTRIPE)],
                    den_o.at[pl.ds(c * PADN + r0, NSTRIPE)])
    pltpu.sync_copy(cnt_sh.at[pl.ds(s * 3 * NSTRIPE, 3 * NSTRIPE)],
                    cnt_o.at[pl.ds(c * 3 * PADN + s * 3 * NSTRIPE,
                                   3 * NSTRIPE)])


def _phase_a(eintl, hsd, x2, zr, zs):
    ib = lambda: pltpu.VMEM((CHUNK,), jnp.int32)
    fb = lambda: pltpu.VMEM((CHUNK,), jnp.float32)
    e3 = lambda: pltpu.VMEM((3 * CHUNK,), jnp.int32)
    dma = pltpu.SemaphoreType.DMA
    f = _mesh(
        _pa_body,
        out_type=[
            jax.ShapeDtypeStruct((2 * PADN, D), jnp.float32),
            jax.ShapeDtypeStruct((2 * PADN,), jnp.float32),
            jax.ShapeDtypeStruct((2 * 3 * PADN,), jnp.float32),
        ],
        scratch_types=[
            [e3(), e3()],
            [ib(), ib()], [ib(), ib()],
            [fb(), fb()], [fb(), fb()], [fb(), fb()],
            [ib(), ib()],
            fb(),
            [pltpu.VMEM((CHUNK, D), jnp.float32),
             pltpu.VMEM((CHUNK, D), jnp.float32)],
            pltpu.VMEM_SHARED((PADN, D), jnp.float32),
            pltpu.VMEM_SHARED((PADN,), jnp.float32),
            pltpu.VMEM_SHARED((3 * PADN,), jnp.float32),
            [dma, dma], [dma, dma], [dma, dma], [dma, dma],
        ],
    )
    return f(eintl, hsd, x2, zr, zs)


def _seg_body(npass, ei_h, tab_h, z64_h, sums_o,
              eb, gidxb, sidxb, rows, acc, esem, sem, ssem):
    c = lax.axis_index("c")
    s = lax.axis_index("s")
    z0 = s * (3 * N // 16)
    nch = (E // 16) // CHUNK
    gc0 = s * nch

    def load_edges(j, b):
        pltpu.async_copy(ei_h.at[pl.ds((gc0 + j) * 3 * CHUNK, 3 * CHUNK)],
                         eb[b], esem[b])

    for p in range(npass):
        qoff = (c * npass + p) * PADN

        pltpu.sync_copy(z64_h.at[pl.ds(z0, 3 * N // 16)],
                        acc.at[pl.ds(z0, 3 * N // 16)])
        plsc.subcore_barrier()

        def prep_issue(b, qo):
            pltpu.make_async_copy(
                ei_h.at[pl.ds(0, 3 * CHUNK)], eb[b], esem[b]).wait()
            for k in range(CHUNK // 16):
                sl = pl.ds(k * 16, 16)
                gidxb[b][sl] = qo + eb[b][sl]
                sidxb[b][sl] = (eb[b][pl.ds(2 * CHUNK + k * 16, 16)] * N
                                + eb[b][pl.ds(CHUNK + k * 16, 16)])
            pltpu.async_copy(tab_h.at[gidxb[b]], rows[b], sem[b])

        def wait_in(b):
            pltpu.make_async_copy(tab_h.at[gidxb[b]], rows[b], sem[b]).wait()

        def scat(b):
            pltpu.async_copy(rows[b], acc.at[sidxb[b]], ssem[b], add=True)

        def wait_sc(b):
            pltpu.make_async_copy(rows[b], acc.at[sidxb[b]], ssem[b]).wait()

        load_edges(0, 0)
        load_edges(1, 1)
        prep_issue(0, qoff)

        def body(i, carry):
            j0 = i * 2

            @pl.when(i > 0)
            def _():
                wait_sc(1)

            @pl.when(j0 + 1 < nch)
            def _():
                prep_issue(1, qoff)

            @pl.when(j0 + 2 < nch)
            def _():
                load_edges(j0 + 2, 0)

            wait_in(0)
            scat(0)

            @pl.when(j0 + 2 < nch)
            def _():
                wait_sc(0)
                prep_issue(0, qoff)

            @pl.when(j0 + 3 < nch)
            def _():
                load_edges(j0 + 3, 1)

            @pl.when(j0 + 1 < nch)
            def _():
                wait_in(1)
                scat(1)

            return carry

        lax.fori_loop(0, (nch + 1) // 2, body, 0)
        wait_sc(0)
        if nch % 2 == 0:
            wait_sc(1)
        plsc.subcore_barrier()

        for r in range(R):
            pltpu.sync_copy(
                acc.at[pl.ds(r * N + s * (N // 16), N // 16)],
                sums_o.at[pl.ds((c * npass + p) * 3 * PADN + r * PADN
                                + s * (N // 16), N // 16)])
        if p + 1 < npass:
            plsc.subcore_barrier()


def _phase_seg(npass, eintl, tab, z64):
    ib = lambda: pltpu.VMEM((CHUNK,), jnp.int32)
    e3 = lambda: pltpu.VMEM((3 * CHUNK,), jnp.int32)
    dma = pltpu.SemaphoreType.DMA
    f = _mesh(
        functools.partial(_seg_body, npass),
        out_type=jax.ShapeDtypeStruct((2 * npass * 3 * PADN, 64),
                                      jnp.float32),
        scratch_types=[
            [e3(), e3()],
            [ib(), ib()], [ib(), ib()],
            [pltpu.VMEM((CHUNK, 64), jnp.float32),
             pltpu.VMEM((CHUNK, 64), jnp.float32)],
            pltpu.VMEM_SHARED((3 * N, 64), jnp.float32),
            [dma, dma], [dma, dma], [dma, dma],
        ],
    )
    return f(eintl, tab, z64)


def _tcm_body(x2, a0, a1, d_r, htab_o, h64_o):
    pid = pl.program_id(0)
    agg = a0[...] + a1[...]
    d0 = d_r[pl.ds(pid * TCROWS, TCROWS)]
    d1 = d_r[pl.ds(PADN + pid * TCROWS, TCROWS)]
    den = (d0 + d1).reshape(TCROWS, 1)
    htab = x2[...] + agg / (den + 1e-16)
    htab_o[...] = htab
    h64_o[0] = htab[:, :64]
    h64_o[1] = htab[:, 64:]


def _tc_mid(x2, aggp, denp):
    row = lambda i: (i, 0)
    return pl.pallas_call(
        _tcm_body,
        grid=(TCBLK,),
        in_specs=[
            pl.BlockSpec((TCROWS, D), row),
            pl.BlockSpec((TCROWS, D), row),
            pl.BlockSpec((TCROWS, D), lambda i: (TCBLK + i, 0)),
            pl.BlockSpec((2 * PADN,), lambda i: (0,)),
        ],
        out_specs=[
            pl.BlockSpec((TCROWS, D), row),
            pl.BlockSpec((2, TCROWS, 64), lambda i: (0, i, 0)),
        ],
        out_shape=[
            jax.ShapeDtypeStruct((PADN, D), jnp.float32),
            jax.ShapeDtypeStruct((2, PADN, 64), jnp.float32),
        ],
    )(x2, aggp, aggp, denp)


def _tc1_body(htab, s1, c_r, wroot1, wr1, bg1, gg1, beg1, wroot2, bg2,
              z64_o, root2_o):
    pid = pl.program_id(0)
    cnt = (c_r[0:3, pl.ds(pid * TCROWS, TCROWS)]
           + c_r[3:6, pl.ds(pid * TCROWS, TCROWS)])
    inv = (1.0 / jnp.maximum(cnt, 1.0))[:, :, None]
    s1v = s1[...]
    means = jnp.concatenate([s1v[0:3], s1v[3:6]], axis=2) * inv
    w1v = wr1[...]
    agg1 = (means[0] @ w1v[0] + means[1] @ w1v[1] + means[2] @ w1v[2])
    z = htab[...] @ wroot1[...] + bg1[...] + agg1
    z1 = jax.nn.relu(z * _BN * gg1[...] + beg1[...])
    root2_o[...] = z1 @ wroot2[...] + bg2[...]
    for q in range(4):
        z64_o[q] = z1[:, q * 64:(q + 1) * 64]


def _tc1(htab, sums1, cntp6, wroot1, wr1, bg1, gg1, beg1, wroot2, bg2):
    row = lambda i: (i, 0)
    vec = lambda i: (0,)
    return pl.pallas_call(
        _tc1_body,
        grid=(TCBLK,),
        in_specs=[
            pl.BlockSpec((TCROWS, D), row),
            pl.BlockSpec((6, TCROWS, 64), lambda i: (0, i, 0)),
            pl.BlockSpec((2 * R, PADN), lambda i: (0, 0)),
            pl.BlockSpec((D, H), lambda i: (0, 0)),
            pl.BlockSpec((R, D, H), lambda i: (0, 0, 0)),
            pl.BlockSpec((H,), vec),
            pl.BlockSpec((H,), vec),
            pl.BlockSpec((H,), vec),
            pl.BlockSpec((H, OUT), lambda i: (0, 0)),
            pl.BlockSpec((OUT,), vec),
        ],
        out_specs=[
            pl.BlockSpec((4, TCROWS, 64), lambda i: (0, i, 0)),
            pl.BlockSpec((TCROWS, OUT), row),
        ],
        out_shape=[
            jax.ShapeDtypeStruct((4, PADN, 64), jnp.float32),
            jax.ShapeDtypeStruct((PADN, OUT), jnp.float32),
        ],
    )(htab, sums1, cntp6, wroot1, wr1, bg1, gg1, beg1, wroot2, bg2)


def _tc2_body(x2, root2, s2, c_r, wr2, gg2, beg2,
              w1, b1, g1, be1, w2, b2, g2, be2, wout, bout, out_o):
    pid = pl.program_id(0)
    cnt = (c_r[0:3, pl.ds(pid * TCROWS, TCROWS)]
           + c_r[3:6, pl.ds(pid * TCROWS, TCROWS)])
    inv = (1.0 / jnp.maximum(cnt, 1.0))[:, :, None]
    s2v = s2[...]
    means = jnp.concatenate([s2v[0:3], s2v[3:6], s2v[6:9], s2v[9:12]],
                            axis=2) * inv
    w2v = wr2[...]
    agg2 = (means[0] @ w2v[0] + means[1] @ w2v[1] + means[2] @ w2v[2])
    z2 = jax.nn.relu((root2[...] + agg2) * _BN * gg2[...] + beg2[...])
    zm = x2[...] @ w1[...] + b1[...]
    zm = jax.nn.relu(zm * _BN * g1[...] + be1[...])
    zm = zm @ w2[...] + b2[...]
    zm = jax.nn.relu(zm * _BN * g2[...] + be2[...])
    wa = wout[...][:H]
    wb = wout[...][H:]
    out_o[...] = zm @ wa + z2 @ wb + bout[...]


def _tc2(x2, root2, sums2, cntp6, wr2, gg2, beg2,
         w1, b1, g1, be1, w2, b2, g2, be2, wout, bout):
    row = lambda i: (i, 0)
    vec = lambda i: (0,)
    out = pl.pallas_call(
        _tc2_body,
        grid=(TCBLK,),
        in_specs=[
            pl.BlockSpec((TCROWS, D), row),
            pl.BlockSpec((TCROWS, OUT), row),
            pl.BlockSpec((12, TCROWS, 64), lambda i: (0, i, 0)),
            pl.BlockSpec((2 * R, PADN), lambda i: (0, 0)),
            pl.BlockSpec((R, H, OUT), lambda i: (0, 0, 0)),
            pl.BlockSpec((OUT,), vec),
            pl.BlockSpec((OUT,), vec),
            pl.BlockSpec((D, H), lambda i: (0, 0)),
            pl.BlockSpec((H,), vec),
            pl.BlockSpec((H,), vec),
            pl.BlockSpec((H,), vec),
            pl.BlockSpec((H, OUT), lambda i: (0, 0)),
            pl.BlockSpec((OUT,), vec),
            pl.BlockSpec((OUT,), vec),
            pl.BlockSpec((OUT,), vec),
            pl.BlockSpec((2 * OUT, 1), lambda i: (0, 0)),
            pl.BlockSpec((1,), vec),
        ],
        out_specs=pl.BlockSpec((TCROWS, 1), row),
        out_shape=jax.ShapeDtypeStruct((PADN, 1), jnp.float32),
    )(x2, root2, sums2, cntp6, wr2, gg2, beg2, w1, b1, g1, be1,
      w2, b2, g2, be2, wout, bout)
    return out[:N, 0]


def kernel(x, edge_index, edge_type, hsd, W1, b1, g1, be1, W2, b2, g2, be2,
           Wroot1, Wr1, bg1, gg1, beg1, Wroot2, Wr2, bg2, gg2, beg2,
           Wout, bout, Wp1, bp1, Wp2, bp2):
    eintl = jnp.stack([edge_index[0].reshape(-1, CHUNK),
                       edge_index[1].reshape(-1, CHUNK),
                       edge_type.reshape(-1, CHUNK)],
                      axis=1).reshape(3 * E)

    x2 = jnp.zeros((PADN, D), jnp.float32).at[:N].set(x)
    zr = jnp.zeros((PADN, 128), jnp.float32)
    zs = jnp.zeros((3 * PADN,), jnp.float32)
    zq = jnp.zeros((3 * PADN, 64), jnp.float32)

    aggp, denp, cntp = _phase_a(eintl, hsd, x2, zr, zs)
    cntp6 = cntp.reshape(2 * R, PADN)

    htab, h64 = _tc_mid(x2, aggp, denp)

    sums1 = _phase_seg(1, eintl, h64.reshape(2 * PADN, 64), zq)

    ztab, root2 = _tc1(htab, sums1.reshape(6, PADN, 64), cntp6,
                       Wroot1, Wr1, bg1, gg1, beg1, Wroot2, bg2)

    sums2 = _phase_seg(2, eintl, ztab.reshape(4 * PADN, 64), zq)

    return _tc2(x2, root2, sums2.reshape(12, PADN, 64), cntp6, Wr2,
                gg2, beg2, W1, b1, g1, be1, W2, b2, g2, be2, Wout, bout)


# speedup vs baseline: 20.6212x; 1.1669x over previous
import functools

import jax
import jax.numpy as jnp
from jax import lax
from jax.experimental import pallas as pl
from jax.experimental.pallas import tpu as pltpu
from jax.experimental.pallas import tpu_sc as plsc

N = 10000
E = 320000
D = 128
H = 256
OUT = 256
R = 3
TAU = 0.1

_BN = float(1.0 / (1.0 + 1e-5) ** 0.5)

PADN = 10240
CHUNK = 80
NSTRIPE = PADN // 16
TCROWS = 1280
TCBLK = PADN // TCROWS

_mesh = functools.partial(
    pl.kernel,
    mesh=plsc.VectorSubcoreMesh(core_axis_name="c", subcore_axis_name="s"),
    compiler_params=pltpu.CompilerParams(use_tc_tiling_on_sc=False),
)


def _pa_body(ei_h, hsd_h, x2_h, zr_h, zs_h,
             agg_o, den_o, cnt_o,
             eb, srcb, dstb, sv, dv, escb, sidxb, onesb, rows,
             acc, den_sh, cnt_sh, esem, sem, sem2, ssem):
    c = lax.axis_index("c")
    s = lax.axis_index("s")
    r0 = s * NSTRIPE
    nch = (E // 32) // CHUNK
    gc0 = (c * 16 + s) * nch

    pltpu.sync_copy(zr_h.at[pl.ds(r0, NSTRIPE)], acc.at[pl.ds(r0, NSTRIPE)])
    pltpu.sync_copy(zs_h.at[pl.ds(r0, NSTRIPE)], den_sh.at[pl.ds(r0, NSTRIPE)])
    pltpu.sync_copy(zs_h.at[pl.ds(s * 3 * NSTRIPE, 3 * NSTRIPE)],
                    cnt_sh.at[pl.ds(s * 3 * NSTRIPE, 3 * NSTRIPE)])
    for k in range(CHUNK // 16):
        onesb[pl.ds(k * 16, 16)] = jnp.ones((16,), jnp.float32)
    plsc.subcore_barrier()

    def load_edges(j, b):
        pltpu.async_copy(ei_h.at[pl.ds((gc0 + j) * 3 * CHUNK, 3 * CHUNK)],
                         eb[b], esem[b])

    def prep_issue(b):
        pltpu.make_async_copy(
            ei_h.at[pl.ds(0, 3 * CHUNK)], eb[b], esem[b]).wait()
        for k in range(CHUNK // 16):
            sl = pl.ds(k * 16, 16)
            srcv = eb[b][sl]
            dstv = eb[b][pl.ds(CHUNK + k * 16, 16)]
            srcb[b][sl] = srcv
            dstb[b][sl] = dstv
            sidxb[b][sl] = eb[b][pl.ds(2 * CHUNK + k * 16, 16)] * PADN + dstv
        pltpu.async_copy(hsd_h.at[srcb[b]], sv[b], sem[b])
        pltpu.async_copy(hsd_h.at[dstb[b]], dv[b], sem[b])
        pltpu.async_copy(x2_h.at[srcb[b]], rows[b], sem2[b])

    def wait_in(b):
        pltpu.make_async_copy(hsd_h.at[srcb[b]], sv[b], sem[b]).wait()
        pltpu.make_async_copy(hsd_h.at[dstb[b]], dv[b], sem[b]).wait()
        pltpu.make_async_copy(x2_h.at[srcb[b]], rows[b], sem2[b]).wait()

    def process(b):
        for k in range(CHUNK // 16):
            sl = pl.ds(k * 16, 16)
            esc = jnp.exp(jnp.abs(sv[b][sl] - dv[b][sl]) * (-1.0 / TAU))
            escb[b][sl] = esc
        pltpu.sync_copy(escb[b], den_sh.at[dstb[b]], add=True)
        pltpu.sync_copy(onesb, cnt_sh.at[sidxb[b]], add=True)
        for kg in range(CHUNK // 16):
            ev = escb[b][pl.ds(kg * 16, 16)]
            for jl in range(16):
                j = kg * 16 + jl
                e = ev[jl]
                for m in range(D // 16):
                    cs = pl.ds(m * 16, 16)
                    rows[b][j, cs] = rows[b][j, cs] * e
        pltpu.async_copy(rows[b], acc.at[dstb[b]], ssem[b], add=True)

    def wait_sc(b):
        pltpu.make_async_copy(rows[b], acc.at[dstb[b]], ssem[b]).wait()

    load_edges(0, 0)
    load_edges(1, 1)
    prep_issue(0)

    def body(i, carry):
        j0 = i * 2

        @pl.when(i > 0)
        def _():
            wait_sc(1)

        @pl.when(j0 + 1 < nch)
        def _():
            prep_issue(1)

        @pl.when(j0 + 2 < nch)
        def _():
            load_edges(j0 + 2, 0)

        wait_in(0)
        process(0)

        @pl.when(j0 + 2 < nch)
        def _():
            wait_sc(0)
            prep_issue(0)

        @pl.when(j0 + 3 < nch)
        def _():
            load_edges(j0 + 3, 1)

        @pl.when(j0 + 1 < nch)
        def _():
            wait_in(1)
            process(1)

        return carry

    lax.fori_loop(0, (nch + 1) // 2, body, 0)
    wait_sc(0)
    if nch % 2 == 0:
        wait_sc(1)
    plsc.subcore_barrier()

    pltpu.sync_copy(acc.at[pl.ds(r0, NSTRIPE)],
                    agg_o.at[pl.ds(c * PADN + r0, NSTRIPE)])
    pltpu.sync_copy(den_sh.at[pl.ds(r0, NSTRIPE)],
                    den_o.at[pl.ds(c * PADN + r0, NSTRIPE)])
    pltpu.sync_copy(cnt_sh.at[pl.ds(s * 3 * NSTRIPE, 3 * NSTRIPE)],
                    cnt_o.at[pl.ds(c * 3 * PADN + s * 3 * NSTRIPE,
                                   3 * NSTRIPE)])


def _phase_a(eintl, hsd, x2, zr, zs):
    ib = lambda: pltpu.VMEM((CHUNK,), jnp.int32)
    fb = lambda: pltpu.VMEM((CHUNK,), jnp.float32)
    e3 = lambda: pltpu.VMEM((3 * CHUNK,), jnp.int32)
    dma = pltpu.SemaphoreType.DMA
    f = _mesh(
        _pa_body,
        out_type=[
            jax.ShapeDtypeStruct((2 * PADN, D), jnp.float32),
            jax.ShapeDtypeStruct((2 * PADN,), jnp.float32),
            jax.ShapeDtypeStruct((2 * 3 * PADN,), jnp.float32),
        ],
        scratch_types=[
            [e3(), e3()],
            [ib(), ib()], [ib(), ib()],
            [fb(), fb()], [fb(), fb()], [fb(), fb()],
            [ib(), ib()],
            fb(),
            [pltpu.VMEM((CHUNK, D), jnp.float32),
             pltpu.VMEM((CHUNK, D), jnp.float32)],
            pltpu.VMEM_SHARED((PADN, D), jnp.float32),
            pltpu.VMEM_SHARED((PADN,), jnp.float32),
            pltpu.VMEM_SHARED((3 * PADN,), jnp.float32),
            [dma, dma], [dma, dma], [dma, dma], [dma, dma],
        ],
    )
    return f(eintl, hsd, x2, zr, zs)


def _seg_body(npass, ei_h, tab_h, z64_h, sums_o,
              eb, gidxb, sidxb, rows, acc, esem, sem, ssem):
    c = lax.axis_index("c")
    s = lax.axis_index("s")
    z0 = s * (3 * N // 16)
    nch = (E // 16) // CHUNK
    gc0 = s * nch

    def load_edges(j, b):
        pltpu.async_copy(ei_h.at[pl.ds((gc0 + j) * 3 * CHUNK, 3 * CHUNK)],
                         eb[b], esem[b])

    for p in range(npass):
        qoff = (c * npass + p) * PADN

        pltpu.sync_copy(z64_h.at[pl.ds(z0, 3 * N // 16)],
                        acc.at[pl.ds(z0, 3 * N // 16)])
        plsc.subcore_barrier()

        def prep_issue(b, qo):
            pltpu.make_async_copy(
                ei_h.at[pl.ds(0, 3 * CHUNK)], eb[b], esem[b]).wait()
            for k in range(CHUNK // 16):
                sl = pl.ds(k * 16, 16)
                gidxb[b][sl] = qo + eb[b][sl]
                sidxb[b][sl] = (eb[b][pl.ds(2 * CHUNK + k * 16, 16)] * N
                                + eb[b][pl.ds(CHUNK + k * 16, 16)])
            pltpu.async_copy(tab_h.at[gidxb[b]], rows[b], sem[b])

        def wait_in(b):
            pltpu.make_async_copy(tab_h.at[gidxb[b]], rows[b], sem[b]).wait()

        def scat(b):
            pltpu.async_copy(rows[b], acc.at[sidxb[b]], ssem[b], add=True)

        def wait_sc(b):
            pltpu.make_async_copy(rows[b], acc.at[sidxb[b]], ssem[b]).wait()

        load_edges(0, 0)
        load_edges(1, 1)
        prep_issue(0, qoff)

        def body(i, carry):
            j0 = i * 2

            @pl.when(i > 0)
            def _():
                wait_sc(1)

            @pl.when(j0 + 1 < nch)
            def _():
                prep_issue(1, qoff)

            @pl.when(j0 + 2 < nch)
            def _():
                load_edges(j0 + 2, 0)

            wait_in(0)
            scat(0)

            @pl.when(j0 + 2 < nch)
            def _():
                wait_sc(0)
                prep_issue(0, qoff)

            @pl.when(j0 + 3 < nch)
            def _():
                load_edges(j0 + 3, 1)

            @pl.when(j0 + 1 < nch)
            def _():
                wait_in(1)
                scat(1)

            return carry

        lax.fori_loop(0, (nch + 1) // 2, body, 0)
        wait_sc(0)
        if nch % 2 == 0:
            wait_sc(1)
        plsc.subcore_barrier()

        for r in range(R):
            pltpu.sync_copy(
                acc.at[pl.ds(r * N + s * (N // 16), N // 16)],
                sums_o.at[pl.ds((c * npass + p) * 3 * PADN + r * PADN
                                + s * (N // 16), N // 16)])
        if p + 1 < npass:
            plsc.subcore_barrier()


def _phase_seg(npass, eintl, tab, z64):
    ib = lambda: pltpu.VMEM((CHUNK,), jnp.int32)
    e3 = lambda: pltpu.VMEM((3 * CHUNK,), jnp.int32)
    dma = pltpu.SemaphoreType.DMA
    f = _mesh(
        functools.partial(_seg_body, npass),
        out_type=jax.ShapeDtypeStruct((2 * npass * 3 * PADN, 64),
                                      jnp.float32),
        scratch_types=[
            [e3(), e3()],
            [ib(), ib()], [ib(), ib()],
            [pltpu.VMEM((CHUNK, 64), jnp.float32),
             pltpu.VMEM((CHUNK, 64), jnp.float32)],
            pltpu.VMEM_SHARED((3 * N, 64), jnp.float32),
            [dma, dma], [dma, dma], [dma, dma],
        ],
    )
    return f(eintl, tab, z64)


def _tcm_body(x2, a0, a1, d_r, htab_o, h64_o):
    pid = pl.program_id(0)
    agg = a0[...] + a1[...]
    d0 = d_r[pl.ds(pid * TCROWS, TCROWS)]
    d1 = d_r[pl.ds(PADN + pid * TCROWS, TCROWS)]
    den = (d0 + d1).reshape(TCROWS, 1)
    htab = x2[...] + agg / (den + 1e-16)
    htab_o[...] = htab
    h64_o[0] = htab[:, :64]
    h64_o[1] = htab[:, 64:]


def _tc_mid(x2, aggp, denp):
    row = lambda i: (i, 0)
    return pl.pallas_call(
        _tcm_body,
        grid=(TCBLK,),
        in_specs=[
            pl.BlockSpec((TCROWS, D), row),
            pl.BlockSpec((TCROWS, D), row),
            pl.BlockSpec((TCROWS, D), lambda i: (TCBLK + i, 0)),
            pl.BlockSpec((2 * PADN,), lambda i: (0,)),
        ],
        out_specs=[
            pl.BlockSpec((TCROWS, D), row),
            pl.BlockSpec((2, TCROWS, 64), lambda i: (0, i, 0)),
        ],
        out_shape=[
            jax.ShapeDtypeStruct((PADN, D), jnp.float32),
            jax.ShapeDtypeStruct((2, PADN, 64), jnp.float32),
        ],
    )(x2, aggp, aggp, denp)


def _tc1_body(htab, s1, c_r, wroot1, wr1, bg1, gg1, beg1, wroot2, bg2, wr2,
              ztab_o, invc_o, root2_o):
    pid = pl.program_id(0)
    cnt = (c_r[0:3, pl.ds(pid * TCROWS, TCROWS)]
           + c_r[3:6, pl.ds(pid * TCROWS, TCROWS)])
    invm = 1.0 / jnp.maximum(cnt, 1.0)
    inv = invm[:, :, None]
    s1v = s1[...]
    means = jnp.concatenate([s1v[0:3], s1v[3:6]], axis=2) * inv
    w1v = wr1[...]
    agg1 = (means[0] @ w1v[0] + means[1] @ w1v[1] + means[2] @ w1v[2])
    z = htab[...] @ wroot1[...] + bg1[...] + agg1
    z1 = jax.nn.relu(z * _BN * gg1[...] + beg1[...])
    root2_o[...] = z1 @ wroot2[...] + bg2[...]
    invc_o[...] = invm
    w2v = wr2[...]
    for r in range(R):
        zt = z1 @ w2v[r]
        ztab_o[r] = zt[:, :128]
        ztab_o[R + r] = zt[:, 128:]


def _tc1(htab, sums1, cntp6, wroot1, wr1, bg1, gg1, beg1, wroot2, bg2, wr2):
    row = lambda i: (i, 0)
    vec = lambda i: (0,)
    return pl.pallas_call(
        _tc1_body,
        grid=(TCBLK,),
        in_specs=[
            pl.BlockSpec((TCROWS, D), row),
            pl.BlockSpec((6, TCROWS, 64), lambda i: (0, i, 0)),
            pl.BlockSpec((2 * R, PADN), lambda i: (0, 0)),
            pl.BlockSpec((D, H), lambda i: (0, 0)),
            pl.BlockSpec((R, D, H), lambda i: (0, 0, 0)),
            pl.BlockSpec((H,), vec),
            pl.BlockSpec((H,), vec),
            pl.BlockSpec((H,), vec),
            pl.BlockSpec((H, OUT), lambda i: (0, 0)),
            pl.BlockSpec((OUT,), vec),
            pl.BlockSpec((R, H, OUT), lambda i: (0, 0, 0)),
        ],
        out_specs=[
            pl.BlockSpec((2 * R, TCROWS, 128), lambda i: (0, i, 0)),
            pl.BlockSpec((R, TCROWS), lambda i: (0, i)),
            pl.BlockSpec((TCROWS, OUT), row),
        ],
        out_shape=[
            jax.ShapeDtypeStruct((2 * R, PADN, 128), jnp.float32),
            jax.ShapeDtypeStruct((R, PADN), jnp.float32),
            jax.ShapeDtypeStruct((PADN, OUT), jnp.float32),
        ],
    )(htab, sums1, cntp6, wroot1, wr1, bg1, gg1, beg1, wroot2, bg2, wr2)


def _pw_body(ei_h, tab_h, invc_h, zr_h, agg_o,
             eb, dstb, gidxb, widxb, wv, rows, acc,
             esem, sem, sem2, ssem):
    c = lax.axis_index("c")
    s = lax.axis_index("s")
    r0 = s * NSTRIPE
    nch = (E // 16) // CHUNK
    gc0 = s * nch
    qoff = c * 3 * PADN

    pltpu.sync_copy(zr_h.at[pl.ds(r0, NSTRIPE)], acc.at[pl.ds(r0, NSTRIPE)])
    plsc.subcore_barrier()

    def load_edges(j, b):
        pltpu.async_copy(ei_h.at[pl.ds((gc0 + j) * 3 * CHUNK, 3 * CHUNK)],
                         eb[b], esem[b])

    def prep_issue(b):
        pltpu.make_async_copy(
            ei_h.at[pl.ds(0, 3 * CHUNK)], eb[b], esem[b]).wait()
        for k in range(CHUNK // 16):
            sl = pl.ds(k * 16, 16)
            dstv = eb[b][pl.ds(CHUNK + k * 16, 16)]
            tv = eb[b][pl.ds(2 * CHUNK + k * 16, 16)] * PADN
            dstb[b][sl] = dstv
            gidxb[b][sl] = (qoff + tv) + eb[b][sl]
            widxb[b][sl] = tv + dstv
        pltpu.async_copy(tab_h.at[gidxb[b]], rows[b], sem[b])
        pltpu.async_copy(invc_h.at[widxb[b]], wv[b], sem2[b])

    def wait_in(b):
        pltpu.make_async_copy(tab_h.at[gidxb[b]], rows[b], sem[b]).wait()
        pltpu.make_async_copy(invc_h.at[widxb[b]], wv[b], sem2[b]).wait()

    def process(b):
        for kg in range(CHUNK // 16):
            ev = wv[b][pl.ds(kg * 16, 16)]
            for jl in range(16):
                j = kg * 16 + jl
                e = ev[jl]
                for m in range(128 // 16):
                    cs = pl.ds(m * 16, 16)
                    rows[b][j, cs] = rows[b][j, cs] * e
        pltpu.async_copy(rows[b], acc.at[dstb[b]], ssem[b], add=True)

    def wait_sc(b):
        pltpu.make_async_copy(rows[b], acc.at[dstb[b]], ssem[b]).wait()

    load_edges(0, 0)
    load_edges(1, 1)
    prep_issue(0)

    def body(i, carry):
        j0 = i * 2

        @pl.when(i > 0)
        def _():
            wait_sc(1)

        @pl.when(j0 + 1 < nch)
        def _():
            prep_issue(1)

        @pl.when(j0 + 2 < nch)
        def _():
            load_edges(j0 + 2, 0)

        wait_in(0)
        process(0)

        @pl.when(j0 + 2 < nch)
        def _():
            wait_sc(0)
            prep_issue(0)

        @pl.when(j0 + 3 < nch)
        def _():
            load_edges(j0 + 3, 1)

        @pl.when(j0 + 1 < nch)
        def _():
            wait_in(1)
            process(1)

        return carry

    lax.fori_loop(0, (nch + 1) // 2, body, 0)
    wait_sc(0)
    if nch % 2 == 0:
        wait_sc(1)
    plsc.subcore_barrier()

    pltpu.sync_copy(acc.at[pl.ds(r0, NSTRIPE)],
                    agg_o.at[pl.ds(c * PADN + r0, NSTRIPE)])


def _phase_w(eintl, ztab, invc, zr):
    ib = lambda: pltpu.VMEM((CHUNK,), jnp.int32)
    fb = lambda: pltpu.VMEM((CHUNK,), jnp.float32)
    e3 = lambda: pltpu.VMEM((3 * CHUNK,), jnp.int32)
    dma = pltpu.SemaphoreType.DMA
    f = _mesh(
        _pw_body,
        out_type=jax.ShapeDtypeStruct((2 * PADN, 128), jnp.float32),
        scratch_types=[
            [e3(), e3()],
            [ib(), ib()], [ib(), ib()], [ib(), ib()],
            [fb(), fb()],
            [pltpu.VMEM((CHUNK, 128), jnp.float32),
             pltpu.VMEM((CHUNK, 128), jnp.float32)],
            pltpu.VMEM_SHARED((PADN, 128), jnp.float32),
            [dma, dma], [dma, dma], [dma, dma], [dma, dma],
        ],
    )
    return f(eintl, ztab, invc, zr)


def _tc2_body(x2, root2, aggw, gg2, beg2,
              w1, b1, g1, be1, w2, b2, g2, be2, wout, bout, out_o):
    av = aggw[...]
    agg2 = jnp.concatenate([av[0], av[1]], axis=1)
    z2 = jax.nn.relu((root2[...] + agg2) * _BN * gg2[...] + beg2[...])
    zm = x2[...] @ w1[...] + b1[...]
    zm = jax.nn.relu(zm * _BN * g1[...] + be1[...])
    zm = zm @ w2[...] + b2[...]
    zm = jax.nn.relu(zm * _BN * g2[...] + be2[...])
    wa = wout[...][:H]
    wb = wout[...][H:]
    out_o[...] = zm @ wa + z2 @ wb + bout[...]


def _tc2(x2, root2, aggw, gg2, beg2,
         w1, b1, g1, be1, w2, b2, g2, be2, wout, bout):
    row = lambda i: (i, 0)
    vec = lambda i: (0,)
    out = pl.pallas_call(
        _tc2_body,
        grid=(TCBLK,),
        in_specs=[
            pl.BlockSpec((TCROWS, D), row),
            pl.BlockSpec((TCROWS, OUT), row),
            pl.BlockSpec((2, TCROWS, 128), lambda i: (0, i, 0)),
            pl.BlockSpec((OUT,), vec),
            pl.BlockSpec((OUT,), vec),
            pl.BlockSpec((D, H), lambda i: (0, 0)),
            pl.BlockSpec((H,), vec),
            pl.BlockSpec((H,), vec),
            pl.BlockSpec((H,), vec),
            pl.BlockSpec((H, OUT), lambda i: (0, 0)),
            pl.BlockSpec((OUT,), vec),
            pl.BlockSpec((OUT,), vec),
            pl.BlockSpec((OUT,), vec),
            pl.BlockSpec((2 * OUT, 1), lambda i: (0, 0)),
            pl.BlockSpec((1,), vec),
        ],
        out_specs=pl.BlockSpec((TCROWS, 1), row),
        out_shape=jax.ShapeDtypeStruct((PADN, 1), jnp.float32),
    )(x2, root2, aggw, gg2, beg2, w1, b1, g1, be1,
      w2, b2, g2, be2, wout, bout)
    return out[:N, 0]


def kernel(x, edge_index, edge_type, hsd, W1, b1, g1, be1, W2, b2, g2, be2,
           Wroot1, Wr1, bg1, gg1, beg1, Wroot2, Wr2, bg2, gg2, beg2,
           Wout, bout, Wp1, bp1, Wp2, bp2):
    eintl = jnp.stack([edge_index[0].reshape(-1, CHUNK),
                       edge_index[1].reshape(-1, CHUNK),
                       edge_type.reshape(-1, CHUNK)],
                      axis=1).reshape(3 * E)

    x2 = jnp.zeros((PADN, D), jnp.float32).at[:N].set(x)
    zr = jnp.zeros((PADN, 128), jnp.float32)
    zs = jnp.zeros((3 * PADN,), jnp.float32)
    zq = jnp.zeros((3 * PADN, 64), jnp.float32)

    aggp, denp, cntp = _phase_a(eintl, hsd, x2, zr, zs)
    cntp6 = cntp.reshape(2 * R, PADN)

    htab, h64 = _tc_mid(x2, aggp, denp)

    sums1 = _phase_seg(1, eintl, h64.reshape(2 * PADN, 64), zq)

    ztab, invc, root2 = _tc1(htab, sums1.reshape(6, PADN, 64), cntp6,
                             Wroot1, Wr1, bg1, gg1, beg1, Wroot2, bg2, Wr2)

    aggw = _phase_w(eintl, ztab.reshape(2 * R * PADN, 128),
                    invc.reshape(R * PADN), zr)

    return _tc2(x2, root2, aggw.reshape(2, PADN, 128),
                gg2, beg2, W1, b1, g1, be1, W2, b2, g2, be2, Wout, bout)
